# baseline (speedup 1.0000x reference)
import functools

import jax
import jax.numpy as jnp
from jax import lax
from jax.experimental import pallas as pl
from jax.experimental.pallas import tpu as pltpu
from jax.experimental.pallas import tpu_sc as plsc

D = 128
_B, _N, _K = 512, 50, 16
_NC, _NS = 2, 16
_NW = _NC * _NS
_SPW = _B // _NW
_GCH = 8
_GW = (_N * _K) // _GCH
_GRID_A = 8
_BB = 64
_RB = _BB * _N


def _mm_relu_body(x_ref, w_ref, o_ref):
    o_ref[...] = jnp.maximum(
        jnp.dot(x_ref[...], w_ref[...], preferred_element_type=jnp.float32), 0.0)


def _stage_a(we_flat, W_self, interpret=False):
    rows = we_flat.shape[0]
    grid_a = max(1, rows // (3200 * 2))
    blk = rows // grid_a
    return pl.pallas_call(
        _mm_relu_body,
        grid=(grid_a,),
        in_specs=[pl.BlockSpec((blk, D), lambda i: (i, 0)),
                  pl.BlockSpec((D, D), lambda i: (0, 0))],
        out_specs=pl.BlockSpec((blk, D), lambda i: (i, 0)),
        out_shape=jax.ShapeDtypeStruct((rows, D), jnp.float32),
        interpret=interpret,
    )(we_flat, W_self)


def _sc_agg(h0, nb):
    bchunk = h0.shape[0]
    spw = bchunk // _NW
    mesh = plsc.VectorSubcoreMesh(core_axis_name="c", subcore_axis_name="s")

    @functools.partial(
        pl.kernel,
        mesh=mesh,
        out_type=jax.ShapeDtypeStruct((bchunk, _N, D), jnp.float32),
        scratch_types=[
            pltpu.VMEM((2, _N, D), jnp.float32),
            pltpu.VMEM((2, _N, _K), jnp.int32),
            pltpu.VMEM((_N, D), jnp.float32),
            pltpu.SemaphoreType.DMA,
            pltpu.SemaphoreType.DMA,
        ],
    )
    def agg(h0_hbm, nb_hbm, a_hbm, tbl, idx, a_t, sem_t, sem_i):
        wid = lax.axis_index("s") * _NC + lax.axis_index("c")
        b0 = wid * spw

        def copies(si, buf):
            return (pltpu.make_async_copy(h0_hbm.at[b0 + si], tbl.at[buf], sem_t),
                    pltpu.make_async_copy(nb_hbm.at[b0 + si], idx.at[buf], sem_i))

        for cp in copies(0, 0):
            cp.start()

        def sample_body(si, carry):
            buf = lax.rem(si, 2)
            for cp in copies(si, buf):
                cp.wait()

            @pl.when(si + 1 < spw)
            def _():
                for cp in copies(si + 1, 1 - buf):
                    cp.start()

            def n_body(n, c2):
                vidx = idx[buf, n, :]
                accs = [None] * (D // 16)
                for k in range(_K):
                    row = vidx[k]
                    for cc in range(D // 16):
                        v = tbl[buf, row, pl.ds(cc * 16, 16)]
                        accs[cc] = v if k == 0 else accs[cc] + v
                for cc in range(D // 16):
                    a_t[n, pl.ds(cc * 16, 16)] = accs[cc]
                return c2

            lax.fori_loop(0, _N, n_body, 0, unroll=2)
            pltpu.sync_copy(a_t, a_hbm.at[b0 + si])
            return carry

        lax.fori_loop(0, spw, sample_body, 0)

    return agg(h0, nb)


def _stage_c_body(we_ref, a_ref, m_ref, ws_ref, wn_ref, w2_ref, o_ref):
    h = jnp.maximum(
        jnp.dot(we_ref[...], ws_ref[...], preferred_element_type=jnp.float32)
        + jnp.dot(a_ref[...], wn_ref[...],
                  preferred_element_type=jnp.float32) * (1.0 / _K),
        0.0)
    r = lax.broadcasted_iota(jnp.int32, (_BB, _RB), 1)
    s = lax.broadcasted_iota(jnp.int32, (_BB, _RB), 0)
    m = m_ref[...].reshape(1, _RB)
    sel = jnp.where(r // _N == s, 1.0, 0.0) * m
    pooled = jnp.dot(sel, h, preferred_element_type=jnp.float32)
    o_ref[...] = jnp.dot(pooled, w2_ref[...], preferred_element_type=jnp.float32)


def _stage_c(we_flat, a_flat, mask_rows, W_self, W_neigh, weight2, interpret=False):
    grid = mask_rows.shape[0]
    bout = grid * _BB
    return pl.pallas_call(
        _stage_c_body,
        grid=(grid,),
        in_specs=[pl.BlockSpec((_RB, D), lambda i: (i, 0)),
                  pl.BlockSpec((_RB, D), lambda i: (i, 0)),
                  pl.BlockSpec((1, 1, _RB), lambda i: (i, 0, 0)),
                  pl.BlockSpec((D, D), lambda i: (0, 0)),
                  pl.BlockSpec((D, D), lambda i: (0, 0)),
                  pl.BlockSpec((D, D), lambda i: (0, 0))],
        out_specs=pl.BlockSpec((_BB, D), lambda i: (i, 0)),
        out_shape=jax.ShapeDtypeStruct((bout, D), jnp.float32),
        interpret=interpret,
    )(we_flat, a_flat, mask_rows, W_self, W_neigh, weight2)


_NCHUNK = 2


def kernel(word_embs, neibors, mask, W_self, W_neigh, weight2):
    nb = neibors.astype(jnp.int32)
    bc = _B // _NCHUNK
    we_full = word_embs.reshape(_B * _N, D)
    h0_full = _stage_a(we_full, W_self).reshape(_B, _N, D)
    aggs = [_sc_agg(h0_full[c * bc:(c + 1) * bc], nb[c * bc:(c + 1) * bc])
            for c in range(_NCHUNK)]
    outs = []
    for c in range(_NCHUNK):
        sl = slice(c * bc, (c + 1) * bc)
        we_flat = word_embs[sl].reshape(bc * _N, D)
        mask_rows = mask[sl].reshape(bc // _BB, 1, _RB)
        a_flat = aggs[c].reshape(bc * _N, D)
        outs.append(_stage_c(we_flat, a_flat, mask_rows, W_self, W_neigh, weight2))
    return jnp.concatenate(outs, axis=0)


# speedup vs baseline: 1.1074x; 1.1074x over previous
import functools

import jax
import jax.numpy as jnp
from jax import lax
from jax.experimental import pallas as pl
from jax.experimental.pallas import tpu as pltpu
from jax.experimental.pallas import tpu_sc as plsc

D = 128
_B, _N, _K = 512, 50, 16
_NC, _NS = 2, 16
_NW = _NC * _NS
_SPW = _B // _NW
_GCH = 8
_GW = (_N * _K) // _GCH
_GRID_A = 8
_BB = 64
_RB = _BB * _N


def _mm_relu_body(x_ref, w_ref, o_ref):
    o_ref[...] = jnp.maximum(
        jnp.dot(x_ref[...], w_ref[...], preferred_element_type=jnp.float32), 0.0)


def _stage_a(we_flat, W_self, interpret=False):
    rows = we_flat.shape[0]
    grid_a = max(1, rows // (3200 * 2))
    blk = rows // grid_a
    return pl.pallas_call(
        _mm_relu_body,
        grid=(grid_a,),
        in_specs=[pl.BlockSpec((blk, D), lambda i: (i, 0)),
                  pl.BlockSpec((D, D), lambda i: (0, 0))],
        out_specs=pl.BlockSpec((blk, D), lambda i: (i, 0)),
        out_shape=jax.ShapeDtypeStruct((rows, D), jnp.float32),
        interpret=interpret,
    )(we_flat, W_self)


def _sc_agg(h0, nb):
    bchunk = h0.shape[0]
    spw = bchunk // _NW
    mesh = plsc.VectorSubcoreMesh(core_axis_name="c", subcore_axis_name="s")

    @functools.partial(
        pl.kernel,
        mesh=mesh,
        out_type=jax.ShapeDtypeStruct((bchunk, _N, D), jnp.float32),
        scratch_types=[
            pltpu.VMEM((2, _N, D), jnp.float32),
            pltpu.VMEM((2, _N, _K), jnp.int32),
            pltpu.VMEM((2, _N, D), jnp.float32),
            pltpu.SemaphoreType.DMA,
            pltpu.SemaphoreType.DMA,
            pltpu.SemaphoreType.DMA,
            pltpu.SemaphoreType.DMA,
        ],
    )
    def agg(h0_hbm, nb_hbm, a_hbm, tbl, idx, a_t, sem_t, sem_i, sem_a0, sem_a1):
        wid = lax.axis_index("s") * _NC + lax.axis_index("c")
        b0 = wid * spw

        def copies(si, buf):
            return (pltpu.make_async_copy(h0_hbm.at[b0 + si], tbl.at[buf], sem_t),
                    pltpu.make_async_copy(nb_hbm.at[b0 + si], idx.at[buf], sem_i))

        def out_copy(si, buf, sem):
            return pltpu.make_async_copy(a_t.at[buf], a_hbm.at[b0 + si], sem)

        for cp in copies(0, 0):
            cp.start()

        def sample_body(si, carry):
            buf = lax.rem(si, 2)
            for cp in copies(si, buf):
                cp.wait()

            @pl.when(si + 1 < spw)
            def _():
                for cp in copies(si + 1, 1 - buf):
                    cp.start()

            @pl.when(si >= 2)
            def _():
                @pl.when(buf == 0)
                def _():
                    out_copy(si - 2, 0, sem_a0).wait()

                @pl.when(buf == 1)
                def _():
                    out_copy(si - 2, 1, sem_a1).wait()

            def n_body(n, c2):
                vidx = idx[buf, n, :]
                accs = [None] * (D // 16)
                for k in range(_K):
                    row = vidx[k]
                    for cc in range(D // 16):
                        v = tbl[buf, row, pl.ds(cc * 16, 16)]
                        accs[cc] = v if k == 0 else accs[cc] + v
                for cc in range(D // 16):
                    a_t[buf, n, pl.ds(cc * 16, 16)] = accs[cc]
                return c2

            lax.fori_loop(0, _N, n_body, 0, unroll=2)

            @pl.when(buf == 0)
            def _():
                out_copy(si, 0, sem_a0).start()

            @pl.when(buf == 1)
            def _():
                out_copy(si, 1, sem_a1).start()

            return carry

        lax.fori_loop(0, spw, sample_body, 0)
        out_copy(spw - 2, 0, sem_a0).wait()
        out_copy(spw - 1, 1, sem_a1).wait()

    return agg(h0, nb)


def _stage_c_body(we_ref, a_ref, m_ref, ws_ref, wn_ref, w2_ref, o_ref):
    h = jnp.maximum(
        jnp.dot(we_ref[...], ws_ref[...], preferred_element_type=jnp.float32)
        + jnp.dot(a_ref[...], wn_ref[...],
                  preferred_element_type=jnp.float32) * (1.0 / _K),
        0.0)
    r = lax.broadcasted_iota(jnp.int32, (_BB, _RB), 1)
    s = lax.broadcasted_iota(jnp.int32, (_BB, _RB), 0)
    m = m_ref[...].reshape(1, _RB)
    sel = jnp.where(r // _N == s, 1.0, 0.0) * m
    pooled = jnp.dot(sel, h, preferred_element_type=jnp.float32)
    o_ref[...] = jnp.dot(pooled, w2_ref[...], preferred_element_type=jnp.float32)


def _stage_c(we_flat, a_flat, mask_rows, W_self, W_neigh, weight2, interpret=False):
    grid = mask_rows.shape[0]
    bout = grid * _BB
    return pl.pallas_call(
        _stage_c_body,
        grid=(grid,),
        in_specs=[pl.BlockSpec((_RB, D), lambda i: (i, 0)),
                  pl.BlockSpec((_RB, D), lambda i: (i, 0)),
                  pl.BlockSpec((1, 1, _RB), lambda i: (i, 0, 0)),
                  pl.BlockSpec((D, D), lambda i: (0, 0)),
                  pl.BlockSpec((D, D), lambda i: (0, 0)),
                  pl.BlockSpec((D, D), lambda i: (0, 0))],
        out_specs=pl.BlockSpec((_BB, D), lambda i: (i, 0)),
        out_shape=jax.ShapeDtypeStruct((bout, D), jnp.float32),
        interpret=interpret,
    )(we_flat, a_flat, mask_rows, W_self, W_neigh, weight2)


_NCHUNK = 2


def kernel(word_embs, neibors, mask, W_self, W_neigh, weight2):
    nb = neibors.astype(jnp.int32)
    bc = _B // _NCHUNK
    outs = []
    for c in range(_NCHUNK):
        sl = slice(c * bc, (c + 1) * bc)
        we_flat = word_embs[sl].reshape(bc * _N, D)
        mask_rows = mask[sl].reshape(bc // _BB, 1, _RB)
        h0 = _stage_a(we_flat, W_self).reshape(bc, _N, D)
        a = _sc_agg(h0, nb[sl])
        a_flat = a.reshape(bc * _N, D)
        outs.append(_stage_c(we_flat, a_flat, mask_rows, W_self, W_neigh, weight2))
    return jnp.concatenate(outs, axis=0)


# speedup vs baseline: 1.1936x; 1.0778x over previous
import functools

import jax
import jax.numpy as jnp
from jax import lax
from jax.experimental import pallas as pl
from jax.experimental.pallas import tpu as pltpu
from jax.experimental.pallas import tpu_sc as plsc

D = 128
_B, _N, _K = 512, 50, 16
_NC, _NS = 2, 16
_NW = _NC * _NS
_SPW = _B // _NW
_GCH = 8
_GW = (_N * _K) // _GCH
_GRID_A = 8
_BB = 64
_RB = _BB * _N


def _mm_relu_body(x_ref, w_ref, o_ref):
    o_ref[...] = jnp.maximum(
        jnp.dot(x_ref[...], w_ref[...], preferred_element_type=jnp.float32), 0.0)


def _stage_a(we_flat, W_self, interpret=False):
    rows = we_flat.shape[0]
    grid_a = max(1, rows // (3200 * 2))
    blk = rows // grid_a
    return pl.pallas_call(
        _mm_relu_body,
        grid=(grid_a,),
        in_specs=[pl.BlockSpec((blk, D), lambda i: (i, 0)),
                  pl.BlockSpec((D, D), lambda i: (0, 0))],
        out_specs=pl.BlockSpec((blk, D), lambda i: (i, 0)),
        out_shape=jax.ShapeDtypeStruct((rows, D), jnp.float32),
        interpret=interpret,
    )(we_flat, W_self)


def _sc_agg(h0, nb):
    bchunk = h0.shape[0]
    spw = bchunk // _NW
    mesh = plsc.VectorSubcoreMesh(core_axis_name="c", subcore_axis_name="s")

    @functools.partial(
        pl.kernel,
        mesh=mesh,
        out_type=jax.ShapeDtypeStruct((bchunk, _N, D), jnp.float32),
        scratch_types=[
            pltpu.VMEM((2, _N, D), jnp.float32),
            pltpu.VMEM((2, _N, _K), jnp.int32),
            pltpu.VMEM((2, _N, D), jnp.float32),
            pltpu.SemaphoreType.DMA,
            pltpu.SemaphoreType.DMA,
            pltpu.SemaphoreType.DMA,
            pltpu.SemaphoreType.DMA,
        ],
    )
    def agg(h0_hbm, nb_hbm, a_hbm, tbl, idx, a_t, sem_t, sem_i, sem_a0, sem_a1):
        wid = lax.axis_index("s") * _NC + lax.axis_index("c")
        b0 = wid * spw

        def copies(si, buf):
            return (pltpu.make_async_copy(h0_hbm.at[b0 + si], tbl.at[buf], sem_t),
                    pltpu.make_async_copy(nb_hbm.at[b0 + si], idx.at[buf], sem_i))

        def out_copy(si, buf, sem):
            return pltpu.make_async_copy(a_t.at[buf], a_hbm.at[b0 + si], sem)

        for cp in copies(0, 0):
            cp.start()

        def sample_body(si, carry):
            buf = lax.rem(si, 2)
            for cp in copies(si, buf):
                cp.wait()

            @pl.when(si + 1 < spw)
            def _():
                for cp in copies(si + 1, 1 - buf):
                    cp.start()

            @pl.when(si >= 2)
            def _():
                @pl.when(buf == 0)
                def _():
                    out_copy(si - 2, 0, sem_a0).wait()

                @pl.when(buf == 1)
                def _():
                    out_copy(si - 2, 1, sem_a1).wait()

            def n_body(n, c2):
                vidx = idx[buf, n, :]
                accs = [None] * (D // 16)
                for k in range(_K):
                    row = vidx[k]
                    for cc in range(D // 16):
                        v = tbl[buf, row, pl.ds(cc * 16, 16)]
                        accs[cc] = v if k == 0 else accs[cc] + v
                for cc in range(D // 16):
                    a_t[buf, n, pl.ds(cc * 16, 16)] = accs[cc]
                return c2

            lax.fori_loop(0, _N, n_body, 0, unroll=2)

            @pl.when(buf == 0)
            def _():
                out_copy(si, 0, sem_a0).start()

            @pl.when(buf == 1)
            def _():
                out_copy(si, 1, sem_a1).start()

            return carry

        lax.fori_loop(0, spw, sample_body, 0)
        out_copy(spw - 2, 0, sem_a0).wait()
        out_copy(spw - 1, 1, sem_a1).wait()

    return agg(h0, nb)


def _stage_c_body(we_ref, a_ref, m_ref, ws_ref, wn_ref, w2_ref, o_ref):
    h = jnp.maximum(
        jnp.dot(we_ref[...], ws_ref[...], preferred_element_type=jnp.float32)
        + jnp.dot(a_ref[...], wn_ref[...],
                  preferred_element_type=jnp.float32) * (1.0 / _K),
        0.0)
    r = lax.broadcasted_iota(jnp.int32, (_BB, _RB), 1)
    s = lax.broadcasted_iota(jnp.int32, (_BB, _RB), 0)
    m = m_ref[...].reshape(1, _RB)
    sel = jnp.where(r // _N == s, 1.0, 0.0) * m
    pooled = jnp.dot(sel, h, preferred_element_type=jnp.float32)
    o_ref[...] = jnp.dot(pooled, w2_ref[...], preferred_element_type=jnp.float32)


def _stage_c(we_flat, a_flat, mask_rows, W_self, W_neigh, weight2, interpret=False):
    grid = mask_rows.shape[0]
    bout = grid * _BB
    return pl.pallas_call(
        _stage_c_body,
        grid=(grid,),
        in_specs=[pl.BlockSpec((_RB, D), lambda i: (i, 0)),
                  pl.BlockSpec((_RB, D), lambda i: (i, 0)),
                  pl.BlockSpec((1, 1, _RB), lambda i: (i, 0, 0)),
                  pl.BlockSpec((D, D), lambda i: (0, 0)),
                  pl.BlockSpec((D, D), lambda i: (0, 0)),
                  pl.BlockSpec((D, D), lambda i: (0, 0))],
        out_specs=pl.BlockSpec((_BB, D), lambda i: (i, 0)),
        out_shape=jax.ShapeDtypeStruct((bout, D), jnp.float32),
        interpret=interpret,
    )(we_flat, a_flat, mask_rows, W_self, W_neigh, weight2)


_NCHUNK = 4


def kernel(word_embs, neibors, mask, W_self, W_neigh, weight2):
    nb = neibors.astype(jnp.int32)
    bc = _B // _NCHUNK
    outs = []
    for c in range(_NCHUNK):
        sl = slice(c * bc, (c + 1) * bc)
        we_flat = word_embs[sl].reshape(bc * _N, D)
        mask_rows = mask[sl].reshape(bc // _BB, 1, _RB)
        h0 = _stage_a(we_flat, W_self).reshape(bc, _N, D)
        a = _sc_agg(h0, nb[sl])
        a_flat = a.reshape(bc * _N, D)
        outs.append(_stage_c(we_flat, a_flat, mask_rows, W_self, W_neigh, weight2))
    return jnp.concatenate(outs, axis=0)
